# baseline (speedup 1.0000x reference)
import functools

import jax
import jax.numpy as jnp
from jax import lax
from jax.experimental import pallas as pl
from jax.experimental.pallas import tpu as pltpu
from jax.experimental.pallas import tpu_sc as plsc

N = 100000
D = 128
K = 16
BLOCK_ROWS = 12800
P = BLOCK_ROWS // 8
GRID = (N + BLOCK_ROWS - 1) // BLOCK_ROWS

_F32 = jnp.float32
_DN = (((1,), (1,)), ((), ()))

_SC_ROWS = 640
_SC_PROWS = _SC_ROWS // 8
_SC_NFULL = N // _SC_ROWS
_SC_TAIL_ROWS = N - _SC_NFULL * _SC_ROWS
_SC_TAIL_PROWS = _SC_TAIL_ROWS // 8
_SC_TAIL_PREAD = 24
_PACKED_ROWS = _SC_NFULL * _SC_PROWS + _SC_TAIL_PREAD
_SC_PER_W = 5


def _tc_body(x_ref, c_ref, o_ref):
    c = c_ref[...]
    cm = -2.0 * c
    ones_kd = jnp.ones((K, D), _F32)
    rows = []
    for j in range(8):
        xj = x_ref[pl.Slice(j, P, 8), :]
        s1 = jax.lax.dot_general(cm, xj, _DN, preferred_element_type=_F32)
        s2 = jax.lax.dot_general(ones_kd, xj * xj, _DN,
                                 preferred_element_type=_F32)
        rows.append(s1 + s2)
    u = jnp.concatenate(rows, axis=0)
    b = jnp.tile(1.0 + jnp.sum(c * c, axis=1, keepdims=True), (8, 1))
    t = jnp.maximum(u + b, 1.0)
    r = 1.0 / t
    i2 = lax.broadcasted_iota(jnp.int32, (128, 128), 0)
    j2 = lax.broadcasted_iota(jnp.int32, (128, 128), 1)
    bd = ((i2 // K) == (j2 // K)).astype(_F32)
    s = jax.lax.dot_general(bd, r, (((1,), (0,)), ((), ())),
                            preferred_element_type=_F32)
    qn = r / s
    eye = (i2 == j2).astype(_F32)
    packed = jax.lax.dot_general(qn, eye, (((0,), (0,)), ((), ())),
                                 preferred_element_type=_F32)
    o_ref[...] = packed


def _tc_packed(x, centers):
    return pl.pallas_call(
        _tc_body,
        grid=(GRID,),
        in_specs=[
            pl.BlockSpec((BLOCK_ROWS, D), lambda i: (i, 0)),
            pl.BlockSpec((K, D), lambda i: (0, 0)),
        ],
        out_specs=pl.BlockSpec((P, 128), lambda i: (i, 0)),
        out_shape=jax.ShapeDtypeStruct((_PACKED_ROWS, 128), jnp.float32),
    )(x, centers)


def _sc_place_body(packed_hbm, out_hbm, buf, buf16):
    wid = lax.axis_index("s") * 2 + lax.axis_index("c")

    out3 = out_hbm.reshape(N // 8, 8, K)

    def _repack(p, _):
        for j in range(8):
            buf16[p, j, :] = buf[p, pl.ds(K * j, K)]
        return 0

    for c in range(_SC_PER_W):
        cid = wid + 32 * c

        @pl.when(cid < _SC_NFULL)
        def _copy_chunk():
            pltpu.sync_copy(
                packed_hbm.at[pl.ds(cid * _SC_PROWS, _SC_PROWS), :], buf
            )
            lax.fori_loop(0, _SC_PROWS, _repack, 0)
            pltpu.sync_copy(
                buf16, out3.at[pl.ds(cid * _SC_PROWS, _SC_PROWS), :, :]
            )

        @pl.when(cid == _SC_NFULL)
        def _copy_tail():
            pltpu.sync_copy(
                packed_hbm.at[
                    pl.ds(_SC_NFULL * _SC_PROWS, _SC_TAIL_PREAD), :
                ],
                buf.at[pl.ds(0, _SC_TAIL_PREAD), :],
            )
            lax.fori_loop(0, _SC_TAIL_PROWS, _repack, 0)
            pltpu.sync_copy(
                buf16.at[pl.ds(0, _SC_TAIL_PROWS), :, :],
                out3.at[pl.ds(_SC_NFULL * _SC_PROWS, _SC_TAIL_PROWS), :, :],
            )


_sc_place = functools.partial(
    pl.kernel,
    out_type=jax.ShapeDtypeStruct((N, K), jnp.float32),
    mesh=plsc.VectorSubcoreMesh(core_axis_name="c", subcore_axis_name="s"),
    scratch_types=[
        pltpu.VMEM((_SC_PROWS, 128), jnp.float32),
        pltpu.VMEM((_SC_PROWS, 8, K), jnp.float32),
    ],
)(_sc_place_body)


def kernel(x, centers):
    packed = _tc_packed(x, centers)
    return _sc_place(packed)


# speedup vs baseline: 1.0852x; 1.0852x over previous
import functools

import jax
import jax.numpy as jnp
from jax import lax
from jax.experimental import pallas as pl
from jax.experimental.pallas import tpu as pltpu
from jax.experimental.pallas import tpu_sc as plsc

N = 100000
D = 128
K = 16
BLOCK_ROWS = 12800
P = BLOCK_ROWS // 8
GRID = (N + BLOCK_ROWS - 1) // BLOCK_ROWS

_F32 = jnp.float32
_DN = (((1,), (1,)), ((), ()))

_SC_ROWS = 640
_SC_PROWS = _SC_ROWS // 8
_SC_NFULL = N // _SC_ROWS
_SC_TAIL_ROWS = N - _SC_NFULL * _SC_ROWS
_SC_TAIL_PROWS = _SC_TAIL_ROWS // 8
_SC_TAIL_PREAD = 24
_PACKED_ROWS = _SC_NFULL * _SC_PROWS + _SC_TAIL_PREAD
_SC_PER_W = 5


def _tc_body(x_ref, c_ref, o_ref):
    c = c_ref[...]
    cm = -2.0 * c
    ones_kd = jnp.ones((K, D), _F32)
    rows = []
    for j in range(8):
        xj = x_ref[pl.Slice(j, P, 8), :]
        s1 = jax.lax.dot_general(cm, xj, _DN, preferred_element_type=_F32)
        s2 = jax.lax.dot_general(ones_kd, xj * xj, _DN,
                                 preferred_element_type=_F32)
        rows.append(s1 + s2)
    u = jnp.concatenate(rows, axis=0)
    b = jnp.tile(1.0 + jnp.sum(c * c, axis=1, keepdims=True), (8, 1))
    t = jnp.maximum(u + b, 1.0)
    r = 1.0 / t
    i2 = lax.broadcasted_iota(jnp.int32, (128, 128), 0)
    j2 = lax.broadcasted_iota(jnp.int32, (128, 128), 1)
    bd = ((i2 // K) == (j2 // K)).astype(_F32)
    s = jax.lax.dot_general(bd, r, (((1,), (0,)), ((), ())),
                            preferred_element_type=_F32)
    qn = r / s
    eye = (i2 == j2).astype(_F32)
    packed = jax.lax.dot_general(qn, eye, (((0,), (0,)), ((), ())),
                                 preferred_element_type=_F32)
    o_ref[...] = packed


def _tc_packed(x, centers):
    return pl.pallas_call(
        _tc_body,
        grid=(GRID,),
        in_specs=[
            pl.BlockSpec((BLOCK_ROWS, D), lambda i: (i, 0)),
            pl.BlockSpec((K, D), lambda i: (0, 0)),
        ],
        out_specs=pl.BlockSpec((P, 128), lambda i: (i, 0)),
        out_shape=jax.ShapeDtypeStruct((_PACKED_ROWS, 128), jnp.float32),
    )(x, centers)


def _sc_place_body(packed_hbm, out_hbm, buf, buf16):
    wid = lax.axis_index("s") * 2 + lax.axis_index("c")

    out3 = out_hbm.reshape(N // 8, 8, K)

    def _repack(p, _):
        buf16[p, :, :] = buf[p, :].reshape(8, K)
        return 0

    for c in range(_SC_PER_W):
        cid = wid + 32 * c

        @pl.when(cid < _SC_NFULL)
        def _copy_chunk():
            pltpu.sync_copy(
                packed_hbm.at[pl.ds(cid * _SC_PROWS, _SC_PROWS), :], buf
            )
            lax.fori_loop(0, _SC_PROWS, _repack, 0)
            pltpu.sync_copy(
                buf16, out3.at[pl.ds(cid * _SC_PROWS, _SC_PROWS), :, :]
            )

        @pl.when(cid == _SC_NFULL)
        def _copy_tail():
            pltpu.sync_copy(
                packed_hbm.at[
                    pl.ds(_SC_NFULL * _SC_PROWS, _SC_TAIL_PREAD), :
                ],
                buf.at[pl.ds(0, _SC_TAIL_PREAD), :],
            )
            lax.fori_loop(0, _SC_TAIL_PROWS, _repack, 0)
            pltpu.sync_copy(
                buf16.at[pl.ds(0, _SC_TAIL_PROWS), :, :],
                out3.at[pl.ds(_SC_NFULL * _SC_PROWS, _SC_TAIL_PROWS), :, :],
            )


_sc_place = functools.partial(
    pl.kernel,
    out_type=jax.ShapeDtypeStruct((N, K), jnp.float32),
    mesh=plsc.VectorSubcoreMesh(core_axis_name="c", subcore_axis_name="s"),
    scratch_types=[
        pltpu.VMEM((_SC_PROWS, 128), jnp.float32),
        pltpu.VMEM((_SC_PROWS, 8, K), jnp.float32),
    ],
)(_sc_place_body)


def kernel(x, centers):
    packed = _tc_packed(x, centers)
    return _sc_place(packed)


# speedup vs baseline: 1.1618x; 1.0706x over previous
import functools

import jax
import jax.numpy as jnp
from jax import lax
from jax.experimental import pallas as pl
from jax.experimental.pallas import tpu as pltpu
from jax.experimental.pallas import tpu_sc as plsc

N = 100000
D = 128
K = 16
BLOCK_ROWS = 12800
P = BLOCK_ROWS // 8
GRID = (N + BLOCK_ROWS - 1) // BLOCK_ROWS

_F32 = jnp.float32
_DN = (((1,), (1,)), ((), ()))

_SC_ROWS = 320
_SC_PROWS = _SC_ROWS // 8
_SC_NFULL = N // _SC_ROWS
_SC_TAIL_ROWS = N - _SC_NFULL * _SC_ROWS
_SC_TAIL_PROWS = _SC_TAIL_ROWS // 8
_SC_TAIL_PREAD = 24
_PACKED_ROWS = _SC_NFULL * _SC_PROWS + _SC_TAIL_PREAD
_SC_PER_W = 10


def _tc_body(x_ref, c_ref, o_ref):
    c = c_ref[...]
    cm = -2.0 * c
    ones_kd = jnp.ones((K, D), _F32)
    rows = []
    for j in range(8):
        xj = x_ref[pl.Slice(j, P, 8), :]
        s1 = jax.lax.dot_general(cm, xj, _DN, preferred_element_type=_F32)
        s2 = jax.lax.dot_general(ones_kd, xj * xj, _DN,
                                 preferred_element_type=_F32)
        rows.append(s1 + s2)
    u = jnp.concatenate(rows, axis=0)
    b = jnp.tile(1.0 + jnp.sum(c * c, axis=1, keepdims=True), (8, 1))
    t = jnp.maximum(u + b, 1.0)
    r = 1.0 / t
    i2 = lax.broadcasted_iota(jnp.int32, (128, 128), 0)
    j2 = lax.broadcasted_iota(jnp.int32, (128, 128), 1)
    bd = ((i2 // K) == (j2 // K)).astype(_F32)
    s = jax.lax.dot_general(bd, r, (((1,), (0,)), ((), ())),
                            preferred_element_type=_F32)
    qn = r / s
    eye = (i2 == j2).astype(_F32)
    packed = jax.lax.dot_general(qn, eye, (((0,), (0,)), ((), ())),
                                 preferred_element_type=_F32)
    o_ref[...] = packed


def _tc_packed(x, centers):
    return pl.pallas_call(
        _tc_body,
        grid=(GRID,),
        in_specs=[
            pl.BlockSpec((BLOCK_ROWS, D), lambda i: (i, 0)),
            pl.BlockSpec((K, D), lambda i: (0, 0)),
        ],
        out_specs=pl.BlockSpec((P, 128), lambda i: (i, 0)),
        out_shape=jax.ShapeDtypeStruct((_PACKED_ROWS, 128), jnp.float32),
    )(x, centers)


def _sc_place_body(packed_hbm, out_hbm, bufs, buf16s, rsems, wsems):
    wid = lax.axis_index("s") * 2 + lax.axis_index("c")
    out3 = out_hbm.reshape(N // 8, 8, K)

    def _read(c, slot):
        cid = wid + 32 * c

        @pl.when(cid < _SC_NFULL)
        def _():
            pltpu.make_async_copy(
                packed_hbm.at[pl.ds(cid * _SC_PROWS, _SC_PROWS), :],
                bufs.at[slot], rsems.at[slot],
            ).start()

        @pl.when(cid == _SC_NFULL)
        def _():
            pltpu.make_async_copy(
                packed_hbm.at[pl.ds(_SC_NFULL * _SC_PROWS, _SC_TAIL_PREAD), :],
                bufs.at[slot, pl.ds(0, _SC_TAIL_PREAD), :], rsems.at[slot],
            ).start()

    def _wait_read(c, slot):
        cid = wid + 32 * c

        @pl.when(cid < _SC_NFULL)
        def _():
            pltpu.make_async_copy(
                packed_hbm.at[pl.ds(cid * _SC_PROWS, _SC_PROWS), :],
                bufs.at[slot], rsems.at[slot],
            ).wait()

        @pl.when(cid == _SC_NFULL)
        def _():
            pltpu.make_async_copy(
                packed_hbm.at[pl.ds(_SC_NFULL * _SC_PROWS, _SC_TAIL_PREAD), :],
                bufs.at[slot, pl.ds(0, _SC_TAIL_PREAD), :], rsems.at[slot],
            ).wait()

    def _wcopy(c, slot):
        cid = wid + 32 * c
        full = pltpu.make_async_copy(
            buf16s.at[slot],
            out3.at[pl.ds(cid * _SC_PROWS, _SC_PROWS), :, :], wsems.at[slot],
        )
        tail = pltpu.make_async_copy(
            buf16s.at[slot, pl.ds(0, _SC_TAIL_PROWS), :, :],
            out3.at[pl.ds(_SC_NFULL * _SC_PROWS, _SC_TAIL_PROWS), :, :],
            wsems.at[slot],
        )
        return full, tail

    def _repack_chunk(c, slot):
        cid = wid + 32 * c
        nb = jnp.where(cid < _SC_NFULL, _SC_PROWS, _SC_TAIL_PROWS)

        def _repack(p, _):
            buf16s[slot, p, :, :] = bufs[slot, p, :].reshape(8, K)
            return 0

        @pl.when(cid <= _SC_NFULL)
        def _():
            lax.fori_loop(0, nb, _repack, 0)

    _read(0, 0)
    for c in range(_SC_PER_W):
        slot = c % 2
        if c + 1 < _SC_PER_W:
            _read(c + 1, 1 - slot)
        cid = wid + 32 * c
        _wait_read(c, slot)
        _repack_chunk(c, slot)
        full, tail = _wcopy(c, slot)
        if c >= 2:
            pass

        @pl.when(cid < _SC_NFULL)
        def _():
            full.start()

        @pl.when(cid == _SC_NFULL)
        def _():
            tail.start()

    for c in range(_SC_PER_W):
        slot = c % 2
        cid = wid + 32 * c
        full, tail = _wcopy(c, slot)

        @pl.when(cid < _SC_NFULL)
        def _():
            full.wait()

        @pl.when(cid == _SC_NFULL)
        def _():
            tail.wait()


_sc_place = functools.partial(
    pl.kernel,
    out_type=jax.ShapeDtypeStruct((N, K), jnp.float32),
    mesh=plsc.VectorSubcoreMesh(core_axis_name="c", subcore_axis_name="s"),
    scratch_types=[
        pltpu.VMEM((2, _SC_PROWS, 128), jnp.float32),
        pltpu.VMEM((2, _SC_PROWS, 8, K), jnp.float32),
        pltpu.SemaphoreType.DMA((2,)),
        pltpu.SemaphoreType.DMA((2,)),
    ],
)(_sc_place_body)


def kernel(x, centers):
    packed = _tc_packed(x, centers)
    return _sc_place(packed)


# speedup vs baseline: 4.4654x; 3.8436x over previous
import jax
import jax.numpy as jnp
from jax.experimental import pallas as pl

N = 100000
D = 128
K = 16
BLOCK_ROWS = 12800
GRID = (N + BLOCK_ROWS - 1) // BLOCK_ROWS

_F32 = jnp.float32
_DN = (((1,), (1,)), ((), ()))


def _body(x_ref, c_ref, o_ref):
    x = x_ref[...]
    c = c_ref[...]
    cm = -2.0 * c
    s1 = jax.lax.dot_general(cm, x, _DN, preferred_element_type=_F32)
    s2 = jax.lax.dot_general(
        jnp.ones((K, D), _F32), x * x, _DN, preferred_element_type=_F32
    )
    b = 1.0 + jnp.sum(c * c, axis=1, keepdims=True)
    t = jnp.maximum(s1 + s2 + b, 1.0)
    r = 1.0 / t
    s = jax.lax.dot_general(
        jnp.ones((K, K), _F32), r, (((1,), (0,)), ((), ())),
        preferred_element_type=_F32,
    )
    o_ref[...] = r / s


def kernel(x, centers):
    qt = pl.pallas_call(
        _body,
        grid=(GRID,),
        in_specs=[
            pl.BlockSpec((BLOCK_ROWS, D), lambda i: (i, 0)),
            pl.BlockSpec((K, D), lambda i: (0, 0)),
        ],
        out_specs=pl.BlockSpec((K, BLOCK_ROWS), lambda i: (0, i)),
        out_shape=jax.ShapeDtypeStruct((K, N), jnp.float32),
    )(x, centers)
    return qt.T


# speedup vs baseline: 4.6345x; 1.0379x over previous
import jax
import jax.numpy as jnp
from jax.experimental import pallas as pl

N = 100000
D = 128
K = 16
BLOCK_ROWS = 25600
GRID = (N + BLOCK_ROWS - 1) // BLOCK_ROWS

_F32 = jnp.float32
_DN = (((1,), (1,)), ((), ()))


def _body(x_ref, c_ref, o_ref):
    x = x_ref[...]
    c = c_ref[...]
    cm = -2.0 * c
    s1 = jax.lax.dot_general(cm, x, _DN, preferred_element_type=_F32)
    s2 = jax.lax.dot_general(
        jnp.ones((K, D), _F32), x * x, _DN, preferred_element_type=_F32
    )
    b = 1.0 + jnp.sum(c * c, axis=1, keepdims=True)
    t = jnp.maximum(s1 + s2 + b, 1.0)
    r = 1.0 / t
    s = jax.lax.dot_general(
        jnp.ones((K, K), _F32), r, (((1,), (0,)), ((), ())),
        preferred_element_type=_F32,
    )
    o_ref[...] = r / s


def kernel(x, centers):
    qt = pl.pallas_call(
        _body,
        grid=(GRID,),
        in_specs=[
            pl.BlockSpec((BLOCK_ROWS, D), lambda i: (i, 0)),
            pl.BlockSpec((K, D), lambda i: (0, 0)),
        ],
        out_specs=pl.BlockSpec((K, BLOCK_ROWS), lambda i: (0, i)),
        out_shape=jax.ShapeDtypeStruct((K, N), jnp.float32),
    )(x, centers)
    return qt.T
